# baseline (speedup 1.0000x reference)
import functools
import jax
import jax.numpy as jnp
from jax import lax
from jax.experimental import pallas as pl
from jax.experimental.pallas import tpu as pltpu
from jax.experimental.pallas import tpu_sc as plsc

_H = 32
_W = 32
_F = 128
_HW = _H * _W
_C = 2 * _F
_BS = 16
_NW = 32
_CPW = _C // _NW
_BUF = _CPW * _HW


def _sc_body(table_hbm, out_hbm, table_v, buf_v, sem):
    cid = lax.axis_index("c")
    sid = lax.axis_index("s")
    wid = sid * 2 + cid
    ch0 = wid * _CPW
    is_col = ch0 < _F

    pltpu.sync_copy(table_hbm, table_v)

    @pl.when(is_col)
    def _build_col():
        for p in range(_CPW):
            base = (ch0 + p) * 64
            v0 = table_v[pl.ds(base, 16)]
            v1 = table_v[pl.ds(base + 16, 16)]
            for j in range(_HW // 16):
                buf_v[pl.ds(p * _HW + j * 16, 16)] = v0 if j % 2 == 0 else v1

    @pl.when(jnp.logical_not(is_col))
    def _build_row():
        for p in range(_CPW):
            base = (ch0 + p - _F) * 64 + 32
            rv0 = table_v[pl.ds(base, 16)]
            rv1 = table_v[pl.ds(base + 16, 16)]
            for h in range(_H):
                src = rv0 if h < 16 else rv1
                idx = jnp.full((16, 1), h % 16, jnp.int32)
                val = lax.gather(
                    src, idx,
                    lax.GatherDimensionNumbers(
                        offset_dims=(), collapsed_slice_dims=(0,),
                        start_index_map=(0,)),
                    slice_sizes=(1,),
                    mode=lax.GatherScatterMode.PROMISE_IN_BOUNDS)
                buf_v[pl.ds(p * _HW + 2 * h * 16, 16)] = val
                buf_v[pl.ds(p * _HW + (2 * h + 1) * 16, 16)] = val

    copies = []
    for b in range(_BS):
        dst = out_hbm.at[pl.ds(b * _C * _HW + ch0 * _HW, _BUF)]
        copies.append(pltpu.async_copy(buf_v, dst, sem))
    for cp in copies:
        cp.wait()


def _sc_call(table_t):
    mesh = plsc.VectorSubcoreMesh(core_axis_name="c", subcore_axis_name="s")
    kfn = functools.partial(
        pl.kernel,
        mesh=mesh,
        out_type=jax.ShapeDtypeStruct((_BS * _C * _HW,), jnp.float32),
        scratch_types=[
            pltpu.VMEM((_F * 64,), jnp.float32),
            pltpu.VMEM((_BUF,), jnp.float32),
            pltpu.SemaphoreType.DMA,
        ],
    )(_sc_body)
    return kfn(table_t)


def kernel(mask, row_weight, col_weight):
    bs, h, w = mask.shape
    table_t = jnp.concatenate(
        [col_weight[:w], row_weight[:h]], axis=0
    ).T.reshape(-1)
    out = _sc_call(table_t)
    return out.reshape(bs, _C, h, w)


# speedup vs baseline: 10.3388x; 10.3388x over previous
import jax
import jax.numpy as jnp
from jax import lax
from jax.experimental import pallas as pl
from jax.experimental.pallas import tpu as pltpu

_H = 32
_W = 32
_F = 128
_BS = 16


def _pos_body(col_ref, row_ref, out_hbm, pos_ref, sem):
    x = lax.broadcast_in_dim(col_ref[...], (_H, _W, _F), (1, 2))
    y = lax.broadcast_in_dim(row_ref[...], (_H, _W, _F), (0, 2))
    pos_ref[...] = lax.concatenate([x, y], 2)
    copies = [
        pltpu.make_async_copy(pos_ref, out_hbm.at[b], sem.at[b])
        for b in range(_BS)
    ]
    for c in copies:
        c.start()
    for c in copies:
        c.wait()


def kernel(mask, row_weight, col_weight):
    bs, h, w = mask.shape
    out = pl.pallas_call(
        _pos_body,
        in_specs=[
            pl.BlockSpec((_W, _F), lambda: (0, 0)),
            pl.BlockSpec((_H, _F), lambda: (0, 0)),
        ],
        out_specs=pl.BlockSpec(memory_space=pl.ANY),
        out_shape=jax.ShapeDtypeStruct((bs, h, w, 2 * _F), jnp.float32),
        scratch_shapes=[
            pltpu.VMEM((_H, _W, 2 * _F), jnp.float32),
            pltpu.SemaphoreType.DMA((_BS,)),
        ],
    )(col_weight[:w], row_weight[:h])
    return jnp.transpose(out, (0, 3, 1, 2))
